# Initial kernel scaffold; baseline (speedup 1.0000x reference)
import functools

import jax
import jax.numpy as jnp
from jax import lax
from jax.experimental import pallas as pl
from jax.experimental.pallas import tpu as pltpu
from jax.experimental.pallas import tpu_sc as plsc

N_NODES = 10000
D = 128
N_EDGES = 320000

NC = 2
NS = 16
NW = NC * NS

E_PER_TILE = N_EDGES // NW
CHUNK = 80
N_CHUNKS = E_PER_TILE // CHUNK

ROWS_PER_TILE = N_NODES // NS
ZROWS = 125


def _sc_aggregate(x, src, dst):
  mesh = plsc.VectorSubcoreMesh(
      core_axis_name="c", subcore_axis_name="s", num_cores=NC,
      num_subcores=NS)

  @functools.partial(
      pl.kernel,
      out_type=jax.ShapeDtypeStruct((NC, N_NODES, D), jnp.float32),
      mesh=mesh,
      scratch_types=[
          pltpu.VMEM((CHUNK,), jnp.int32),
          pltpu.VMEM((CHUNK,), jnp.int32),
          pltpu.VMEM((CHUNK, D), jnp.float32),
          pltpu.VMEM((CHUNK,), jnp.int32),
          pltpu.VMEM((CHUNK,), jnp.int32),
          pltpu.VMEM((CHUNK, D), jnp.float32),
          pltpu.VMEM((ZROWS, D), jnp.float32),
          pltpu.VMEM_SHARED((N_NODES, D), jnp.float32),
          pltpu.SemaphoreType.DMA,
          pltpu.SemaphoreType.DMA,
      ],
  )
  def agg_kernel(x_hbm, src_hbm, dst_hbm, out_hbm,
                 src_a, dst_a, rows_a, src_b, dst_b, rows_b,
                 zbuf, acc, sem_a, sem_b):
    c = lax.axis_index("c")
    s = lax.axis_index("s")
    wid = s * NC + c
    base_e = wid * E_PER_TILE

    zeros16 = jnp.zeros((16,), jnp.float32)

    def zrow(i, _):
      for j in range(D // 16):
        zbuf[i, pl.ds(j * 16, 16)] = zeros16
      return 0

    lax.fori_loop(0, ZROWS, zrow, 0)
    r0 = s * ROWS_PER_TILE
    for k in range(ROWS_PER_TILE // ZROWS):
      pltpu.sync_copy(zbuf, acc.at[pl.ds(r0 + k * ZROWS, ZROWS), :])
    plsc.subcore_barrier()

    bufs = ((src_a, dst_a, rows_a, sem_a), (src_b, dst_b, rows_b, sem_b))

    def start(i, b):
      src_v, dst_v, rows_v, sem = bufs[b]
      e0 = base_e + i * CHUNK
      pltpu.sync_copy(src_hbm.at[pl.ds(e0, CHUNK)], src_v)
      pltpu.sync_copy(dst_hbm.at[pl.ds(e0, CHUNK)], dst_v)
      pltpu.async_copy(x_hbm.at[src_v], rows_v, sem)

    def drain(b):
      src_v, dst_v, rows_v, sem = bufs[b]
      pltpu.async_copy(x_hbm.at[src_v], rows_v, sem).wait()
      pltpu.sync_copy(rows_v, acc.at[dst_v], add=True)

    start(0, 0)

    def chunk_body(i, _):
      @pl.when(i < N_CHUNKS - 1)
      def _():
        start(i + 1, (i + 1) % 2)

      @pl.when(i % 2 == 0)
      def _():
        drain(0)

      @pl.when(i % 2 == 1)
      def _():
        drain(1)

      return 0

    lax.fori_loop(0, N_CHUNKS, chunk_body, 0)
    plsc.subcore_barrier()

    pltpu.sync_copy(acc.at[pl.ds(r0, ROWS_PER_TILE), :],
                    out_hbm.at[c, pl.ds(r0, ROWS_PER_TILE), :])

  return agg_kernel(x, src, dst)


BLK = 400


def _mlp_block(x_ref, p0_ref, p1_ref, w1_ref, b1_ref, w2_ref, b2_ref,
               out_ref):
  h = x_ref[...] + p0_ref[...] + p1_ref[...]
  h = jnp.dot(h, w1_ref[...], preferred_element_type=jnp.float32)
  h = jnp.maximum(h + b1_ref[...], 0.0)
  out_ref[...] = (
      jnp.dot(h, w2_ref[...], preferred_element_type=jnp.float32)
      + b2_ref[...])


def _mlp(x, p0, p1, W1, b1, W2, b2):
  grid = (N_NODES // BLK,)
  row_spec = pl.BlockSpec((BLK, D), lambda i: (i, 0))
  full = pl.BlockSpec((D, D), lambda i: (0, 0))
  vec = pl.BlockSpec((1, D), lambda i: (0, 0))
  return pl.pallas_call(
      _mlp_block,
      grid=grid,
      in_specs=[row_spec, row_spec, row_spec, full, vec, full, vec],
      out_specs=row_spec,
      out_shape=jax.ShapeDtypeStruct((N_NODES, D), jnp.float32),
  )(x, p0, p1, W1, b1.reshape(1, D), W2, b2.reshape(1, D))


@jax.jit
def kernel(x, edge_index, W1, b1, W2, b2):
  src = edge_index[0].astype(jnp.int32)
  dst = edge_index[1].astype(jnp.int32)
  partials = _sc_aggregate(x, src, dst)
  return _mlp(x, partials[0], partials[1], W1, b1, W2, b2)


# speedup vs baseline: 8.1292x; 8.1292x over previous
import functools

import jax
import jax.numpy as jnp
from jax import lax
from jax.experimental import pallas as pl
from jax.experimental.pallas import tpu as pltpu
from jax.experimental.pallas import tpu_sc as plsc

N_NODES = 10000
D = 128
N_EDGES = 320000

NC = 2
NS = 16
NW = NC * NS

E_PER_TILE = N_EDGES // NW
CHUNK = 80
N_CHUNKS = E_PER_TILE // CHUNK

N_PAD = 10240
ROWS_PER_TILE = N_PAD // NS
ZROWS = 128


def _sc_aggregate(x, src, dst):
  mesh = plsc.VectorSubcoreMesh(
      core_axis_name="c", subcore_axis_name="s", num_cores=NC,
      num_subcores=NS)

  @functools.partial(
      pl.kernel,
      out_type=jax.ShapeDtypeStruct((NC, N_PAD, D), jnp.float32),
      mesh=mesh,
      scratch_types=[
          pltpu.VMEM((CHUNK,), jnp.int32),
          pltpu.VMEM((CHUNK,), jnp.int32),
          pltpu.VMEM((CHUNK, D), jnp.float32),
          pltpu.VMEM((CHUNK,), jnp.int32),
          pltpu.VMEM((CHUNK,), jnp.int32),
          pltpu.VMEM((CHUNK, D), jnp.float32),
          pltpu.VMEM((ZROWS, D), jnp.float32),
          pltpu.VMEM_SHARED((N_PAD, D), jnp.float32),
          pltpu.SemaphoreType.DMA,
          pltpu.SemaphoreType.DMA,
      ],
  )
  def agg_kernel(x_hbm, src_hbm, dst_hbm, out_hbm,
                 src_a, dst_a, rows_a, src_b, dst_b, rows_b,
                 zbuf, acc, sem_a, sem_b):
    c = lax.axis_index("c")
    s = lax.axis_index("s")
    wid = s * NC + c
    base_e = wid * E_PER_TILE

    zeros16 = jnp.zeros((16,), jnp.float32)

    def zrow(i, _):
      for j in range(D // 16):
        zbuf[i, pl.ds(j * 16, 16)] = zeros16
      return 0

    lax.fori_loop(0, ZROWS, zrow, 0)
    r0 = s * ROWS_PER_TILE
    for k in range(ROWS_PER_TILE // ZROWS):
      pltpu.sync_copy(zbuf, acc.at[pl.ds(r0 + k * ZROWS, ZROWS), :])
    plsc.subcore_barrier()

    bufs = ((src_a, dst_a, rows_a, sem_a), (src_b, dst_b, rows_b, sem_b))

    def start(i, b):
      src_v, dst_v, rows_v, sem = bufs[b]
      e0 = base_e + i * CHUNK
      pltpu.sync_copy(src_hbm.at[pl.ds(e0, CHUNK)], src_v)
      pltpu.sync_copy(dst_hbm.at[pl.ds(e0, CHUNK)], dst_v)
      pltpu.async_copy(x_hbm.at[src_v], rows_v, sem)

    def drain(b):
      src_v, dst_v, rows_v, sem = bufs[b]
      pltpu.make_async_copy(x_hbm.at[src_v], rows_v, sem).wait()
      pltpu.sync_copy(rows_v, acc.at[dst_v], add=True)

    start(0, 0)

    def pair_body(p, _):
      start(2 * p + 1, 1)
      drain(0)
      start(2 * p + 2, 0)
      drain(1)
      return 0

    lax.fori_loop(0, (N_CHUNKS - 1) // 2, pair_body, 0)
    drain(0)
    plsc.subcore_barrier()

    pltpu.sync_copy(acc.at[pl.ds(r0, ROWS_PER_TILE), :],
                    out_hbm.at[c, pl.ds(r0, ROWS_PER_TILE), :])

  return agg_kernel(x, src, dst)


BLK = 400


def _mlp_block(x_ref, p0_ref, p1_ref, w1_ref, b1_ref, w2_ref, b2_ref,
               out_ref):
  h = x_ref[...] + p0_ref[...] + p1_ref[...]
  h = jnp.dot(h, w1_ref[...], preferred_element_type=jnp.float32)
  h = jnp.maximum(h + b1_ref[...], 0.0)
  out_ref[...] = (
      jnp.dot(h, w2_ref[...], preferred_element_type=jnp.float32)
      + b2_ref[...])


def _mlp(x, p0, p1, W1, b1, W2, b2):
  grid = (N_NODES // BLK,)
  row_spec = pl.BlockSpec((BLK, D), lambda i: (i, 0))
  full = pl.BlockSpec((D, D), lambda i: (0, 0))
  vec = pl.BlockSpec((1, D), lambda i: (0, 0))
  return pl.pallas_call(
      _mlp_block,
      grid=grid,
      in_specs=[row_spec, row_spec, row_spec, full, vec, full, vec],
      out_specs=row_spec,
      out_shape=jax.ShapeDtypeStruct((N_NODES, D), jnp.float32),
  )(x, p0, p1, W1, b1.reshape(1, D), W2, b2.reshape(1, D))


@jax.jit
def kernel(x, edge_index, W1, b1, W2, b2):
  src = edge_index[0].astype(jnp.int32)
  dst = edge_index[1].astype(jnp.int32)
  partials = _sc_aggregate(x, src, dst)
  return _mlp(x, partials[0], partials[1], W1, b1, W2, b2)


# speedup vs baseline: 10.4213x; 1.2820x over previous
import functools

import jax
import jax.numpy as jnp
from jax import lax
from jax.experimental import pallas as pl
from jax.experimental.pallas import tpu as pltpu
from jax.experimental.pallas import tpu_sc as plsc

N_NODES = 10000
D = 128
N_EDGES = 320000

NC = 2
NS = 16
NW = NC * NS

CHUNK = 128
N_CHUNKS = 80
E_PER_TILE = N_CHUNKS * CHUNK
E_PAD = NW * E_PER_TILE

N_PAD = 10240
ROWS_PER_TILE = N_PAD // NS

NSLOT = 4


def _sc_aggregate(x, src2d, dst3d):
  mesh = plsc.VectorSubcoreMesh(
      core_axis_name="c", subcore_axis_name="s", num_cores=NC,
      num_subcores=NS)

  @functools.partial(
      pl.kernel,
      out_type=jax.ShapeDtypeStruct((NC, N_PAD, D), jnp.float32),
      mesh=mesh,
      scratch_types=[
          pltpu.VMEM((E_PER_TILE,), jnp.int32),
          [pltpu.VMEM((CHUNK,), jnp.int32) for _ in range(NSLOT)],
          [pltpu.VMEM((CHUNK, D), jnp.float32) for _ in range(2)],
          pltpu.VMEM_SHARED((N_PAD, D), jnp.float32),
          pltpu.SemaphoreType.DMA,
          [pltpu.SemaphoreType.DMA for _ in range(NSLOT)],
          [pltpu.SemaphoreType.DMA for _ in range(2)],
          [pltpu.SemaphoreType.DMA for _ in range(2)],
      ],
  )
  def agg_kernel(x_hbm, src_hbm, dst_hbm, out_hbm,
                 src_all, dst_slot, rows, acc,
                 sem_i, sem_d, sem_g, sem_s):
    c = lax.axis_index("c")
    s = lax.axis_index("s")
    wid = s * NC + c

    pltpu.async_copy(src_hbm.at[wid], src_all, sem_i)

    if True:
      zeros16 = jnp.zeros((16,), jnp.float32)

      def zrow(i, _):
        for j in range(D // 16):
          rows[0][i, pl.ds(j * 16, 16)] = zeros16
        return 0

      lax.fori_loop(0, CHUNK, zrow, 0)
      r0 = s * ROWS_PER_TILE
      for k in range(ROWS_PER_TILE // CHUNK):
        pltpu.sync_copy(rows[0], acc.at[pl.ds(r0 + k * CHUNK, CHUNK), :])

      pltpu.make_async_copy(src_hbm.at[wid], src_all, sem_i).wait()
      plsc.subcore_barrier()

      def load_dst(i, sl):
        pltpu.async_copy(dst_hbm.at[wid, i], dst_slot[sl], sem_d[sl])

      def wait_dst(sl):
        pltpu.make_async_copy(
            dst_hbm.at[wid, 0], dst_slot[sl], sem_d[sl]).wait()

      def start_gather(i, rb):
        pltpu.async_copy(
            x_hbm.at[src_all.at[pl.ds(i * CHUNK, CHUNK)]], rows[rb],
            sem_g[rb])

      def wait_gather(rb):
        pltpu.make_async_copy(
            x_hbm.at[src_all.at[pl.ds(0, CHUNK)]], rows[rb],
            sem_g[rb]).wait()

      def start_scatter(rb, sl):
        pltpu.async_copy(rows[rb], acc.at[dst_slot[sl]], sem_s[rb],
                         add=True)

      def wait_scatter(rb):
        pltpu.make_async_copy(rows[rb], acc.at[dst_slot[0]],
                              sem_s[rb]).wait()

      load_dst(0, 0)
      load_dst(1, 1)
      start_gather(0, 0)

      def quad_body(g, _):
        for b in range(NSLOT):
          i = g * NSLOT + b
          rb = b % 2
          wait_gather(rb)
          wait_dst(b)
          start_scatter(rb, b)
          if b == 0:
            @pl.when(g > 0)
            def _():
              wait_scatter(1 - rb)
          else:
            wait_scatter(1 - rb)
          if b == 3:
            @pl.when(g < N_CHUNKS // NSLOT - 1)
            def _():
              start_gather(i + 1, 1 - rb)
          else:
            start_gather(i + 1, 1 - rb)
          if b >= 2:
            @pl.when(g < N_CHUNKS // NSLOT - 1)
            def _():
              load_dst(i + 2, (b + 2) % NSLOT)
          else:
            load_dst(i + 2, (b + 2) % NSLOT)
        return 0

      lax.fori_loop(0, N_CHUNKS // NSLOT, quad_body, 0)
      wait_scatter(1)
      plsc.subcore_barrier()

      pltpu.sync_copy(acc.at[pl.ds(r0, ROWS_PER_TILE), :],
                      out_hbm.at[c, pl.ds(r0, ROWS_PER_TILE), :])

  return agg_kernel(x, src2d, dst3d)


BLK = 400


def _mlp_block(x_ref, p0_ref, p1_ref, w1_ref, b1_ref, w2_ref, b2_ref,
               out_ref):
  h = x_ref[...] + p0_ref[...] + p1_ref[...]
  h = jnp.dot(h, w1_ref[...], preferred_element_type=jnp.float32)
  h = jnp.maximum(h + b1_ref[...], 0.0)
  out_ref[...] = (
      jnp.dot(h, w2_ref[...], preferred_element_type=jnp.float32)
      + b2_ref[...])


def _mlp(x, p0, p1, W1, b1, W2, b2):
  grid = (N_NODES // BLK,)
  row_spec = pl.BlockSpec((BLK, D), lambda i: (i, 0))
  full = pl.BlockSpec((D, D), lambda i: (0, 0))
  vec = pl.BlockSpec((1, D), lambda i: (0, 0))
  return pl.pallas_call(
      _mlp_block,
      grid=grid,
      in_specs=[row_spec, row_spec, row_spec, full, vec, full, vec],
      out_specs=row_spec,
      out_shape=jax.ShapeDtypeStruct((N_NODES, D), jnp.float32),
  )(x, p0, p1, W1, b1.reshape(1, D), W2, b2.reshape(1, D))


@jax.jit
def kernel(x, edge_index, W1, b1, W2, b2):
  src = edge_index[0].astype(jnp.int32)
  dst = edge_index[1].astype(jnp.int32)
  pad = E_PAD - N_EDGES
  pad_iota = jnp.arange(pad, dtype=jnp.int32)
  src_p = jnp.concatenate([src, pad_iota % N_NODES])
  dst_p = jnp.concatenate([dst, N_NODES + pad_iota % (N_PAD - N_NODES)])
  src2d = src_p.reshape(NW, E_PER_TILE)
  dst3d = dst_p.reshape(NW, N_CHUNKS, CHUNK)
  partials = _sc_aggregate(x, src2d, dst3d)
  return _mlp(x, partials[0], partials[1], W1, b1, W2, b2)


# speedup vs baseline: 11.5305x; 1.1064x over previous
import functools

import jax
import jax.numpy as jnp
from jax import lax
from jax.experimental import pallas as pl
from jax.experimental.pallas import tpu as pltpu
from jax.experimental.pallas import tpu_sc as plsc

N_NODES = 10000
D = 128
N_EDGES = 320000

NC = 2
NS = 16
NW = NC * NS

CHUNK = 128
N_CHUNKS = 80
E_PER_TILE = N_CHUNKS * CHUNK
E_PAD = NW * E_PER_TILE

N_PAD = 10240
ROWS_PER_TILE = N_PAD // NS

NSLOT = 4


def _sc_aggregate(x, src2d, dst3d):
  mesh = plsc.VectorSubcoreMesh(
      core_axis_name="c", subcore_axis_name="s", num_cores=NC,
      num_subcores=NS)

  @functools.partial(
      pl.kernel,
      out_type=jax.ShapeDtypeStruct((NC, N_PAD, D), jnp.float32),
      mesh=mesh,
      scratch_types=[
          pltpu.VMEM((E_PER_TILE,), jnp.int32),
          [pltpu.VMEM((CHUNK,), jnp.int32) for _ in range(NSLOT)],
          [pltpu.VMEM((CHUNK, D), jnp.float32) for _ in range(2)],
          pltpu.VMEM_SHARED((N_PAD, D), jnp.float32),
          pltpu.SemaphoreType.DMA,
          [pltpu.SemaphoreType.DMA for _ in range(NSLOT)],
          [pltpu.SemaphoreType.DMA for _ in range(2)],
          [pltpu.SemaphoreType.DMA for _ in range(2)],
      ],
  )
  def agg_kernel(x_hbm, src_hbm, dst_hbm, out_hbm,
                 src_all, dst_slot, rows, acc,
                 sem_i, sem_d, sem_g, sem_s):
    c = lax.axis_index("c")
    s = lax.axis_index("s")
    wid = s * NC + c

    pltpu.async_copy(src_hbm.at[wid], src_all, sem_i)

    if True:
      zeros16 = jnp.zeros((16,), jnp.float32)

      def zrow(i, _):
        for j in range(D // 16):
          rows[0][i, pl.ds(j * 16, 16)] = zeros16
        return 0

      lax.fori_loop(0, CHUNK, zrow, 0)
      r0 = s * ROWS_PER_TILE
      for k in range(ROWS_PER_TILE // CHUNK):
        pltpu.sync_copy(rows[0], acc.at[pl.ds(r0 + k * CHUNK, CHUNK), :])

      pltpu.make_async_copy(src_hbm.at[wid], src_all, sem_i).wait()
      plsc.subcore_barrier()

      def load_dst(i, sl):
        pltpu.async_copy(dst_hbm.at[wid, i], dst_slot[sl], sem_d[sl])

      def wait_dst(sl):
        pltpu.make_async_copy(
            dst_hbm.at[wid, 0], dst_slot[sl], sem_d[sl]).wait()

      def start_gather(i, rb):
        pltpu.async_copy(
            x_hbm.at[src_all.at[pl.ds(i * CHUNK, CHUNK)]], rows[rb],
            sem_g[rb])

      def wait_gather(rb):
        pltpu.make_async_copy(
            x_hbm.at[src_all.at[pl.ds(0, CHUNK)]], rows[rb],
            sem_g[rb]).wait()

      def start_scatter(rb, sl):
        pltpu.async_copy(rows[rb], acc.at[dst_slot[sl]], sem_s[rb],
                         add=True)

      def wait_scatter(rb):
        pltpu.make_async_copy(rows[rb], acc.at[dst_slot[0]],
                              sem_s[rb]).wait()

      load_dst(0, 0)
      load_dst(1, 1)
      start_gather(0, 0)

      def quad_body(g, _):
        for b in range(NSLOT):
          i = g * NSLOT + b
          rb = b % 2
          wait_gather(rb)
          wait_dst(b)
          start_scatter(rb, b)
          if b == 0:
            @pl.when(g > 0)
            def _():
              wait_scatter(1 - rb)
          else:
            wait_scatter(1 - rb)
          if b == 3:
            @pl.when(g < N_CHUNKS // NSLOT - 1)
            def _():
              start_gather(i + 1, 1 - rb)
          else:
            start_gather(i + 1, 1 - rb)
          if b >= 2:
            @pl.when(g < N_CHUNKS // NSLOT - 1)
            def _():
              load_dst(i + 2, (b + 2) % NSLOT)
          else:
            load_dst(i + 2, (b + 2) % NSLOT)
        return 0

      lax.fori_loop(0, N_CHUNKS // NSLOT, quad_body, 0)
      wait_scatter(1)
      plsc.subcore_barrier()

      pltpu.sync_copy(acc.at[pl.ds(r0, ROWS_PER_TILE), :],
                      out_hbm.at[c, pl.ds(r0, ROWS_PER_TILE), :])

  return agg_kernel(x, src2d, dst3d)


BLK = 2000


def _mlp_block(x_ref, p0_ref, p1_ref, w1_ref, b1_ref, w2_ref, b2_ref,
               out_ref):
  h = x_ref[...] + p0_ref[0] + p1_ref[0]
  h = jnp.dot(h, w1_ref[...], preferred_element_type=jnp.float32)
  h = jnp.maximum(h + b1_ref[...], 0.0)
  out_ref[...] = (
      jnp.dot(h, w2_ref[...], preferred_element_type=jnp.float32)
      + b2_ref[...])


def _mlp(x, partials, W1, b1, W2, b2):
  grid = (N_NODES // BLK,)
  row_spec = pl.BlockSpec((BLK, D), lambda i: (i, 0))
  p0_spec = pl.BlockSpec((1, BLK, D), lambda i: (0, i, 0))
  p1_spec = pl.BlockSpec((1, BLK, D), lambda i: (1, i, 0))
  full = pl.BlockSpec((D, D), lambda i: (0, 0))
  vec = pl.BlockSpec((1, D), lambda i: (0, 0))
  return pl.pallas_call(
      _mlp_block,
      grid=grid,
      in_specs=[row_spec, p0_spec, p1_spec, full, vec, full, vec],
      out_specs=row_spec,
      out_shape=jax.ShapeDtypeStruct((N_NODES, D), jnp.float32),
  )(x, partials, partials, W1, b1.reshape(1, D), W2, b2.reshape(1, D))


@jax.jit
def kernel(x, edge_index, W1, b1, W2, b2):
  src = edge_index[0].astype(jnp.int32)
  dst = edge_index[1].astype(jnp.int32)
  pad = E_PAD - N_EDGES
  pad_iota = jnp.arange(pad, dtype=jnp.int32)
  src_p = jnp.concatenate([src, pad_iota % N_NODES])
  dst_p = jnp.concatenate([dst, N_NODES + pad_iota % (N_PAD - N_NODES)])
  src2d = src_p.reshape(NW, E_PER_TILE)
  dst3d = dst_p.reshape(NW, N_CHUNKS, CHUNK)
  partials = _sc_aggregate(x, src2d, dst3d)
  return _mlp(x, partials, W1, b1, W2, b2)


# speedup vs baseline: 12.0659x; 1.0464x over previous
import functools

import jax
import jax.numpy as jnp
from jax import lax
from jax.experimental import pallas as pl
from jax.experimental.pallas import tpu as pltpu
from jax.experimental.pallas import tpu_sc as plsc

N_NODES = 10000
D = 128
N_EDGES = 320000

NC = 2
NS = 16
NW = NC * NS

CHUNK = 64
N_CHUNKS = 160
E_PER_TILE = N_CHUNKS * CHUNK
E_PAD = NW * E_PER_TILE

N_PAD = 10240
ROWS_PER_TILE = N_PAD // NS

NBUF = 4
NSLOT = 8


def _sc_aggregate(x, src2d, dst3d):
  mesh = plsc.VectorSubcoreMesh(
      core_axis_name="c", subcore_axis_name="s", num_cores=NC,
      num_subcores=NS)

  @functools.partial(
      pl.kernel,
      out_type=jax.ShapeDtypeStruct((NC, N_PAD, D), jnp.float32),
      mesh=mesh,
      scratch_types=[
          pltpu.VMEM((E_PER_TILE,), jnp.int32),
          [pltpu.VMEM((CHUNK,), jnp.int32) for _ in range(NSLOT)],
          [pltpu.VMEM((CHUNK, D), jnp.float32) for _ in range(NBUF)],
          pltpu.VMEM_SHARED((N_PAD, D), jnp.float32),
          pltpu.SemaphoreType.DMA,
          [pltpu.SemaphoreType.DMA for _ in range(NSLOT)],
          [pltpu.SemaphoreType.DMA for _ in range(NBUF)],
          [pltpu.SemaphoreType.DMA for _ in range(NBUF)],
      ],
  )
  def agg_kernel(x_hbm, src_hbm, dst_hbm, out_hbm,
                 src_all, dst_slot, rows, acc,
                 sem_i, sem_d, sem_g, sem_s):
    c = lax.axis_index("c")
    s = lax.axis_index("s")
    wid = s * NC + c

    pltpu.async_copy(src_hbm.at[wid], src_all, sem_i)

    if True:
      zeros16 = jnp.zeros((16,), jnp.float32)

      def zrow(i, _):
        for j in range(D // 16):
          rows[0][i, pl.ds(j * 16, 16)] = zeros16
        return 0

      lax.fori_loop(0, CHUNK, zrow, 0)
      r0 = s * ROWS_PER_TILE
      for k in range(ROWS_PER_TILE // CHUNK):
        pltpu.sync_copy(rows[0], acc.at[pl.ds(r0 + k * CHUNK, CHUNK), :])

      pltpu.make_async_copy(src_hbm.at[wid], src_all, sem_i).wait()
      plsc.subcore_barrier()

      def load_dst(i, sl):
        pltpu.async_copy(dst_hbm.at[wid, i], dst_slot[sl], sem_d[sl])

      def wait_dst(sl):
        pltpu.make_async_copy(
            dst_hbm.at[wid, 0], dst_slot[sl], sem_d[sl]).wait()

      def start_gather(i, rb):
        pltpu.async_copy(
            x_hbm.at[src_all.at[pl.ds(i * CHUNK, CHUNK)]], rows[rb],
            sem_g[rb])

      def wait_gather(rb):
        pltpu.make_async_copy(
            x_hbm.at[src_all.at[pl.ds(0, CHUNK)]], rows[rb],
            sem_g[rb]).wait()

      def start_scatter(rb, sl):
        pltpu.async_copy(rows[rb], acc.at[dst_slot[sl]], sem_s[rb],
                         add=True)

      def wait_scatter(rb):
        pltpu.make_async_copy(rows[rb], acc.at[dst_slot[0]],
                              sem_s[rb]).wait()

      for sl in range(6):
        load_dst(sl, sl)
      start_gather(0, 0)
      start_gather(1, 1)

      NG = N_CHUNKS // NSLOT
      LAST = NG - 1

      def oct_body(g, _):
        for b in range(NSLOT):
          i = g * NSLOT + b
          buf = b % NBUF
          buf2 = (b + 2) % NBUF
          wait_gather(buf)
          wait_dst(b)
          start_scatter(buf, b)
          if b < 2:
            @pl.when(g > 0)
            def _():
              wait_scatter(buf2)
            start_gather(i + 2, buf2)
            load_dst(i + 6, (b + 6) % NSLOT)
          elif b < 6:
            wait_scatter(buf2)
            start_gather(i + 2, buf2)
            @pl.when(g < LAST)
            def _():
              load_dst(i + 6, (b + 6) % NSLOT)
          else:
            wait_scatter(buf2)
            @pl.when(g < LAST)
            def _():
              start_gather(i + 2, buf2)
            @pl.when(g < LAST)
            def _():
              load_dst(i + 6, (b + 6) % NSLOT)
        return 0

      lax.fori_loop(0, NG, oct_body, 0)
      wait_scatter(2)
      wait_scatter(3)
      plsc.subcore_barrier()

      pltpu.sync_copy(acc.at[pl.ds(r0, ROWS_PER_TILE), :],
                      out_hbm.at[c, pl.ds(r0, ROWS_PER_TILE), :])

  return agg_kernel(x, src2d, dst3d)


BLK = 2000


def _mlp_block(x_ref, p0_ref, p1_ref, w1_ref, b1_ref, w2_ref, b2_ref,
               out_ref):
  h = x_ref[...] + p0_ref[0] + p1_ref[0]
  h = jnp.dot(h, w1_ref[...], preferred_element_type=jnp.float32)
  h = jnp.maximum(h + b1_ref[...], 0.0)
  out_ref[...] = (
      jnp.dot(h, w2_ref[...], preferred_element_type=jnp.float32)
      + b2_ref[...])


def _mlp(x, partials, W1, b1, W2, b2):
  grid = (N_NODES // BLK,)
  row_spec = pl.BlockSpec((BLK, D), lambda i: (i, 0))
  p0_spec = pl.BlockSpec((1, BLK, D), lambda i: (0, i, 0))
  p1_spec = pl.BlockSpec((1, BLK, D), lambda i: (1, i, 0))
  full = pl.BlockSpec((D, D), lambda i: (0, 0))
  vec = pl.BlockSpec((1, D), lambda i: (0, 0))
  return pl.pallas_call(
      _mlp_block,
      grid=grid,
      in_specs=[row_spec, p0_spec, p1_spec, full, vec, full, vec],
      out_specs=row_spec,
      out_shape=jax.ShapeDtypeStruct((N_NODES, D), jnp.float32),
  )(x, partials, partials, W1, b1.reshape(1, D), W2, b2.reshape(1, D))


@jax.jit
def kernel(x, edge_index, W1, b1, W2, b2):
  src = edge_index[0].astype(jnp.int32)
  dst = edge_index[1].astype(jnp.int32)
  pad = E_PAD - N_EDGES
  pad_iota = jnp.arange(pad, dtype=jnp.int32)
  src_p = jnp.concatenate([src, pad_iota % N_NODES])
  dst_p = jnp.concatenate([dst, N_NODES + pad_iota % (N_PAD - N_NODES)])
  src2d = src_p.reshape(NW, E_PER_TILE)
  dst3d = dst_p.reshape(NW, N_CHUNKS, CHUNK)
  partials = _sc_aggregate(x, src2d, dst3d)
  return _mlp(x, partials, W1, b1, W2, b2)


# speedup vs baseline: 13.8468x; 1.1476x over previous
import functools

import jax
import jax.numpy as jnp
from jax import lax
from jax.experimental import pallas as pl
from jax.experimental.pallas import tpu as pltpu
from jax.experimental.pallas import tpu_sc as plsc

N_NODES = 10000
D = 128
N_EDGES = 320000

NC = 2
NS = 16
NW = NC * NS

CHUNK = 32
N_CHUNKS = 320
E_PER_TILE = N_CHUNKS * CHUNK
E_PAD = NW * E_PER_TILE

N_PAD = 10240
ROWS_PER_TILE = N_PAD // NS

NBUF = 8
NSLOT = 8


def _sc_aggregate(x, src2d, dst3d):
  mesh = plsc.VectorSubcoreMesh(
      core_axis_name="c", subcore_axis_name="s", num_cores=NC,
      num_subcores=NS)

  @functools.partial(
      pl.kernel,
      out_type=jax.ShapeDtypeStruct((NC, N_PAD, D), jnp.float32),
      mesh=mesh,
      scratch_types=[
          pltpu.VMEM((E_PER_TILE,), jnp.int32),
          [pltpu.VMEM((CHUNK,), jnp.int32) for _ in range(NSLOT)],
          [pltpu.VMEM((CHUNK, D), jnp.float32) for _ in range(NBUF)],
          pltpu.VMEM_SHARED((N_PAD, D), jnp.float32),
          pltpu.SemaphoreType.DMA,
          [pltpu.SemaphoreType.DMA for _ in range(NSLOT)],
          [pltpu.SemaphoreType.DMA for _ in range(NBUF)],
          [pltpu.SemaphoreType.DMA for _ in range(NBUF)],
      ],
  )
  def agg_kernel(x_hbm, src_hbm, dst_hbm, out_hbm,
                 src_all, dst_slot, rows, acc,
                 sem_i, sem_d, sem_g, sem_s):
    c = lax.axis_index("c")
    s = lax.axis_index("s")
    wid = s * NC + c

    pltpu.async_copy(src_hbm.at[wid], src_all, sem_i)

    if True:
      zeros16 = jnp.zeros((16,), jnp.float32)

      def zrow(i, _):
        for j in range(D // 16):
          rows[0][i, pl.ds(j * 16, 16)] = zeros16
        return 0

      lax.fori_loop(0, CHUNK, zrow, 0)
      r0 = s * ROWS_PER_TILE
      for k in range(ROWS_PER_TILE // CHUNK):
        pltpu.sync_copy(rows[0], acc.at[pl.ds(r0 + k * CHUNK, CHUNK), :])

      pltpu.make_async_copy(src_hbm.at[wid], src_all, sem_i).wait()
      plsc.subcore_barrier()

      def load_dst(i, sl):
        pltpu.async_copy(dst_hbm.at[wid, i], dst_slot[sl], sem_d[sl])

      def wait_dst(sl):
        pltpu.make_async_copy(
            dst_hbm.at[wid, 0], dst_slot[sl], sem_d[sl]).wait()

      def start_gather(i, rb):
        pltpu.async_copy(
            x_hbm.at[src_all.at[pl.ds(i * CHUNK, CHUNK)]], rows[rb],
            sem_g[rb])

      def wait_gather(rb):
        pltpu.make_async_copy(
            x_hbm.at[src_all.at[pl.ds(0, CHUNK)]], rows[rb],
            sem_g[rb]).wait()

      def start_scatter(rb, sl):
        pltpu.async_copy(rows[rb], acc.at[dst_slot[sl]], sem_s[rb],
                         add=True)

      def wait_scatter(rb):
        pltpu.make_async_copy(rows[rb], acc.at[dst_slot[0]],
                              sem_s[rb]).wait()

      for sl in range(4):
        load_dst(sl, sl)
      for bb in range(5):
        start_gather(bb, bb)

      NG = N_CHUNKS // NSLOT
      LAST = NG - 1

      def oct_body(g, _):
        for b in range(NSLOT):
          i = g * NSLOT + b
          bn = (b + 5) % NBUF
          wait_gather(b)
          wait_dst(b)
          start_scatter(b, b)
          if b >= 3:
            wait_scatter(bn)
          else:
            @pl.when(g > 0)
            def _():
              wait_scatter(bn)
          if b <= 2:
            start_gather(i + 5, bn)
          else:
            @pl.when(g < LAST)
            def _():
              start_gather(i + 5, bn)
          if b <= 3:
            load_dst(i + 4, (b + 4) % NSLOT)
          else:
            @pl.when(g < LAST)
            def _():
              load_dst(i + 4, (b + 4) % NSLOT)
        return 0

      lax.fori_loop(0, NG, oct_body, 0)
      wait_scatter(5)
      wait_scatter(6)
      wait_scatter(7)
      plsc.subcore_barrier()

      pltpu.sync_copy(acc.at[pl.ds(r0, ROWS_PER_TILE), :],
                      out_hbm.at[c, pl.ds(r0, ROWS_PER_TILE), :])

  return agg_kernel(x, src2d, dst3d)


BLK = 2000


def _mlp_block(x_ref, p0_ref, p1_ref, w1_ref, b1_ref, w2_ref, b2_ref,
               out_ref):
  h = x_ref[...] + p0_ref[0] + p1_ref[0]
  h = jnp.dot(h, w1_ref[...], preferred_element_type=jnp.float32)
  h = jnp.maximum(h + b1_ref[...], 0.0)
  out_ref[...] = (
      jnp.dot(h, w2_ref[...], preferred_element_type=jnp.float32)
      + b2_ref[...])


def _mlp(x, partials, W1, b1, W2, b2):
  grid = (N_NODES // BLK,)
  row_spec = pl.BlockSpec((BLK, D), lambda i: (i, 0))
  p0_spec = pl.BlockSpec((1, BLK, D), lambda i: (0, i, 0))
  p1_spec = pl.BlockSpec((1, BLK, D), lambda i: (1, i, 0))
  full = pl.BlockSpec((D, D), lambda i: (0, 0))
  vec = pl.BlockSpec((1, D), lambda i: (0, 0))
  return pl.pallas_call(
      _mlp_block,
      grid=grid,
      in_specs=[row_spec, p0_spec, p1_spec, full, vec, full, vec],
      out_specs=row_spec,
      out_shape=jax.ShapeDtypeStruct((N_NODES, D), jnp.float32),
  )(x, partials, partials, W1, b1.reshape(1, D), W2, b2.reshape(1, D))


@jax.jit
def kernel(x, edge_index, W1, b1, W2, b2):
  src = edge_index[0].astype(jnp.int32)
  dst = edge_index[1].astype(jnp.int32)
  pad = E_PAD - N_EDGES
  pad_iota = jnp.arange(pad, dtype=jnp.int32)
  src_p = jnp.concatenate([src, pad_iota % N_NODES])
  dst_p = jnp.concatenate([dst, N_NODES + pad_iota % (N_PAD - N_NODES)])
  src2d = src_p.reshape(NW, E_PER_TILE)
  dst3d = dst_p.reshape(NW, N_CHUNKS, CHUNK)
  partials = _sc_aggregate(x, src2d, dst3d)
  return _mlp(x, partials, W1, b1, W2, b2)


# speedup vs baseline: 14.3421x; 1.0358x over previous
import functools

import jax
import jax.numpy as jnp
from jax import lax
from jax.experimental import pallas as pl
from jax.experimental.pallas import tpu as pltpu
from jax.experimental.pallas import tpu_sc as plsc

N_NODES = 10000
D = 128
N_EDGES = 320000

NC = 2
NS = 16
NW = NC * NS

E_PER_TILE = N_EDGES // NW
CHUNK = 32
N_CHUNKS = 312
TAIL = E_PER_TILE - N_CHUNKS * CHUNK

N_PAD = 10240
ROWS_PER_TILE = N_PAD // NS

NBUF = 8


def _sc_aggregate(x, src, dst):
  mesh = plsc.VectorSubcoreMesh(
      core_axis_name="c", subcore_axis_name="s", num_cores=NC,
      num_subcores=NS)

  @functools.partial(
      pl.kernel,
      out_type=jax.ShapeDtypeStruct((NC, N_PAD, D), jnp.float32),
      mesh=mesh,
      scratch_types=[
          pltpu.VMEM((E_PER_TILE,), jnp.int32),
          [pltpu.VMEM((CHUNK,), jnp.int32) for _ in range(NBUF)],
          [pltpu.VMEM((CHUNK, D), jnp.float32) for _ in range(NBUF)],
          pltpu.VMEM((TAIL,), jnp.int32),
          pltpu.VMEM((TAIL,), jnp.int32),
          pltpu.VMEM((TAIL, D), jnp.float32),
          pltpu.VMEM_SHARED((N_PAD, D), jnp.float32),
          pltpu.SemaphoreType.DMA,
          pltpu.SemaphoreType.DMA,
          [pltpu.SemaphoreType.DMA for _ in range(NBUF)],
          [pltpu.SemaphoreType.DMA for _ in range(NBUF)],
          [pltpu.SemaphoreType.DMA for _ in range(NBUF)],
      ],
  )
  def agg_kernel(x_hbm, src_hbm, dst_hbm, out_hbm,
                 src_all, dst_slot, rows, tsrc, tdst, trows, acc,
                 sem_i, sem_z, sem_d, sem_g, sem_s):
    c = lax.axis_index("c")
    s = lax.axis_index("s")
    wid = s * NC + c
    base = wid * E_PER_TILE
    tail0 = base + N_CHUNKS * CHUNK

    pltpu.async_copy(src_hbm.at[pl.ds(base, E_PER_TILE)], src_all, sem_i)
    pltpu.async_copy(src_hbm.at[pl.ds(tail0, TAIL)], tsrc, sem_i)
    pltpu.async_copy(dst_hbm.at[pl.ds(tail0, TAIL)], tdst, sem_i)

    zeros16 = jnp.zeros((16,), jnp.float32)

    def zrow(i, _):
      for j in range(D // 16):
        rows[7][i, pl.ds(j * 16, 16)] = zeros16
      return 0

    lax.fori_loop(0, CHUNK, zrow, 0)
    r0 = s * ROWS_PER_TILE
    for k in range(ROWS_PER_TILE // CHUNK):
      pltpu.async_copy(rows[7], acc.at[pl.ds(r0 + k * CHUNK, CHUNK), :],
                       sem_z)

    def load_dst(i, sl):
      pltpu.async_copy(dst_hbm.at[pl.ds(base + i * CHUNK, CHUNK)],
                       dst_slot[sl], sem_d[sl])

    def wait_dst(sl):
      pltpu.make_async_copy(dst_hbm.at[pl.ds(base, CHUNK)],
                            dst_slot[sl], sem_d[sl]).wait()

    def start_gather(i, rb):
      pltpu.async_copy(
          x_hbm.at[src_all.at[pl.ds(i * CHUNK, CHUNK)]], rows[rb],
          sem_g[rb])

    def wait_gather(rb):
      pltpu.make_async_copy(
          x_hbm.at[src_all.at[pl.ds(0, CHUNK)]], rows[rb],
          sem_g[rb]).wait()

    def start_scatter(rb, sl):
      pltpu.async_copy(rows[rb], acc.at[dst_slot[sl]], sem_s[rb],
                       add=True)

    def wait_scatter(rb):
      pltpu.make_async_copy(rows[rb], acc.at[dst_slot[0]],
                            sem_s[rb]).wait()

    pltpu.make_async_copy(src_hbm.at[pl.ds(base, E_PER_TILE)], src_all,
                          sem_i).wait()
    for sl in range(4):
      load_dst(sl, sl)
    for bb in range(5):
      start_gather(bb, bb)

    for k in range(ROWS_PER_TILE // CHUNK):
      pltpu.make_async_copy(rows[7],
                            acc.at[pl.ds(r0, CHUNK), :], sem_z).wait()
    plsc.subcore_barrier()

    NG = N_CHUNKS // NBUF
    LAST = NG - 1

    def oct_body(g, _):
      for b in range(NBUF):
        i = g * NBUF + b
        bn = (b + 5) % NBUF
        wait_gather(b)
        wait_dst(b)
        start_scatter(b, b)
        if b >= 3:
          wait_scatter(bn)
        else:
          @pl.when(g > 0)
          def _():
            wait_scatter(bn)
        if b <= 2:
          start_gather(i + 5, bn)
        else:
          @pl.when(g < LAST)
          def _():
            start_gather(i + 5, bn)
        if b <= 3:
          load_dst(i + 4, (b + 4) % NBUF)
        else:
          @pl.when(g < LAST)
          def _():
            load_dst(i + 4, (b + 4) % NBUF)
      return 0

    lax.fori_loop(0, NG, oct_body, 0)

    pltpu.make_async_copy(src_hbm.at[pl.ds(tail0, TAIL)], tsrc,
                          sem_i).wait()
    pltpu.make_async_copy(dst_hbm.at[pl.ds(tail0, TAIL)], tdst,
                          sem_i).wait()
    pltpu.sync_copy(x_hbm.at[tsrc], trows)
    pltpu.sync_copy(trows, acc.at[tdst], add=True)

    wait_scatter(5)
    wait_scatter(6)
    wait_scatter(7)
    plsc.subcore_barrier()

    pltpu.sync_copy(acc.at[pl.ds(r0, ROWS_PER_TILE), :],
                    out_hbm.at[c, pl.ds(r0, ROWS_PER_TILE), :])

  return agg_kernel(x, src, dst)


BLK = 2000


def _mlp_block(x_ref, p0_ref, p1_ref, w1_ref, b1_ref, w2_ref, b2_ref,
               out_ref):
  h = x_ref[...] + p0_ref[0] + p1_ref[0]
  h = jnp.dot(h, w1_ref[...], preferred_element_type=jnp.float32)
  h = jnp.maximum(h + b1_ref[...], 0.0)
  out_ref[...] = (
      jnp.dot(h, w2_ref[...], preferred_element_type=jnp.float32)
      + b2_ref[...])


def _mlp(x, partials, W1, b1, W2, b2):
  grid = (N_NODES // BLK,)
  row_spec = pl.BlockSpec((BLK, D), lambda i: (i, 0))
  p0_spec = pl.BlockSpec((1, BLK, D), lambda i: (0, i, 0))
  p1_spec = pl.BlockSpec((1, BLK, D), lambda i: (1, i, 0))
  full = pl.BlockSpec((D, D), lambda i: (0, 0))
  vec = pl.BlockSpec((1, D), lambda i: (0, 0))
  return pl.pallas_call(
      _mlp_block,
      grid=grid,
      in_specs=[row_spec, p0_spec, p1_spec, full, vec, full, vec],
      out_specs=row_spec,
      out_shape=jax.ShapeDtypeStruct((N_NODES, D), jnp.float32),
  )(x, partials, partials, W1, b1.reshape(1, D), W2, b2.reshape(1, D))


@jax.jit
def kernel(x, edge_index, W1, b1, W2, b2):
  src = edge_index[0].astype(jnp.int32)
  dst = edge_index[1].astype(jnp.int32)
  partials = _sc_aggregate(x, src, dst)
  return _mlp(x, partials, W1, b1, W2, b2)


# speedup vs baseline: 14.3470x; 1.0003x over previous
import functools

import jax
import jax.numpy as jnp
from jax import lax
from jax.experimental import pallas as pl
from jax.experimental.pallas import tpu as pltpu
from jax.experimental.pallas import tpu_sc as plsc

N_NODES = 10000
D = 128
N_EDGES = 320000

NC = 2
NS = 16
NW = NC * NS

E_PER_TILE = N_EDGES // NW
CHUNK = 32
N_CHUNKS = 312
TAIL = E_PER_TILE - N_CHUNKS * CHUNK

N_PAD = 10240
ROWS_PER_TILE = N_PAD // NS

NBUF = 8


def _sc_aggregate(x, src, dst):
  mesh = plsc.VectorSubcoreMesh(
      core_axis_name="c", subcore_axis_name="s", num_cores=NC,
      num_subcores=NS)

  @functools.partial(
      pl.kernel,
      out_type=jax.ShapeDtypeStruct((NC, N_PAD, D), jnp.float32),
      mesh=mesh,
      compiler_params=pltpu.CompilerParams(use_tc_tiling_on_sc=True),
      scratch_types=[
          pltpu.VMEM((E_PER_TILE,), jnp.int32),
          [pltpu.VMEM((CHUNK,), jnp.int32) for _ in range(NBUF)],
          [pltpu.VMEM((CHUNK, D), jnp.float32) for _ in range(NBUF)],
          pltpu.VMEM((TAIL,), jnp.int32),
          pltpu.VMEM((TAIL,), jnp.int32),
          pltpu.VMEM((TAIL, D), jnp.float32),
          pltpu.VMEM_SHARED((N_PAD, D), jnp.float32),
          pltpu.SemaphoreType.DMA,
          pltpu.SemaphoreType.DMA,
          [pltpu.SemaphoreType.DMA for _ in range(NBUF)],
          [pltpu.SemaphoreType.DMA for _ in range(NBUF)],
          [pltpu.SemaphoreType.DMA for _ in range(NBUF)],
      ],
  )
  def agg_kernel(x_hbm, src_hbm, dst_hbm, out_hbm,
                 src_all, dst_slot, rows, tsrc, tdst, trows, acc,
                 sem_i, sem_z, sem_d, sem_g, sem_s):
    c = lax.axis_index("c")
    s = lax.axis_index("s")
    wid = s * NC + c
    base = wid * E_PER_TILE
    tail0 = base + N_CHUNKS * CHUNK

    pltpu.async_copy(src_hbm.at[pl.ds(base, E_PER_TILE)], src_all, sem_i)
    pltpu.async_copy(src_hbm.at[pl.ds(tail0, TAIL)], tsrc, sem_i)
    pltpu.async_copy(dst_hbm.at[pl.ds(tail0, TAIL)], tdst, sem_i)

    zeros16 = jnp.zeros((16,), jnp.float32)

    def zrow(i, _):
      for j in range(D // 16):
        rows[7][i, pl.ds(j * 16, 16)] = zeros16
      return 0

    lax.fori_loop(0, CHUNK, zrow, 0)
    r0 = s * ROWS_PER_TILE
    for k in range(ROWS_PER_TILE // CHUNK):
      pltpu.async_copy(rows[7], acc.at[pl.ds(r0 + k * CHUNK, CHUNK), :],
                       sem_z)

    def load_dst(i, sl):
      pltpu.async_copy(dst_hbm.at[pl.ds(base + i * CHUNK, CHUNK)],
                       dst_slot[sl], sem_d[sl])

    def wait_dst(sl):
      pltpu.make_async_copy(dst_hbm.at[pl.ds(base, CHUNK)],
                            dst_slot[sl], sem_d[sl]).wait()

    def start_gather(i, rb):
      pltpu.async_copy(
          x_hbm.at[src_all.at[pl.ds(i * CHUNK, CHUNK)]], rows[rb],
          sem_g[rb])

    def wait_gather(rb):
      pltpu.make_async_copy(
          x_hbm.at[src_all.at[pl.ds(0, CHUNK)]], rows[rb],
          sem_g[rb]).wait()

    def start_scatter(rb, sl):
      pltpu.async_copy(rows[rb], acc.at[dst_slot[sl]], sem_s[rb],
                       add=True)

    def wait_scatter(rb):
      pltpu.make_async_copy(rows[rb], acc.at[dst_slot[0]],
                            sem_s[rb]).wait()

    pltpu.make_async_copy(src_hbm.at[pl.ds(base, E_PER_TILE)], src_all,
                          sem_i).wait()
    for sl in range(4):
      load_dst(sl, sl)
    for bb in range(5):
      start_gather(bb, bb)

    for k in range(ROWS_PER_TILE // CHUNK):
      pltpu.make_async_copy(rows[7],
                            acc.at[pl.ds(r0, CHUNK), :], sem_z).wait()
    plsc.subcore_barrier()

    NG = N_CHUNKS // NBUF
    LAST = NG - 1

    def oct_body(g, _):
      for b in range(NBUF):
        i = g * NBUF + b
        bn = (b + 5) % NBUF
        wait_gather(b)
        wait_dst(b)
        start_scatter(b, b)
        if b >= 3:
          wait_scatter(bn)
        else:
          @pl.when(g > 0)
          def _():
            wait_scatter(bn)
        if b <= 2:
          start_gather(i + 5, bn)
        else:
          @pl.when(g < LAST)
          def _():
            start_gather(i + 5, bn)
        if b <= 3:
          load_dst(i + 4, (b + 4) % NBUF)
        else:
          @pl.when(g < LAST)
          def _():
            load_dst(i + 4, (b + 4) % NBUF)
      return 0

    lax.fori_loop(0, NG, oct_body, 0)

    pltpu.make_async_copy(src_hbm.at[pl.ds(tail0, TAIL)], tsrc,
                          sem_i).wait()
    pltpu.make_async_copy(dst_hbm.at[pl.ds(tail0, TAIL)], tdst,
                          sem_i).wait()
    pltpu.sync_copy(x_hbm.at[tsrc], trows)
    pltpu.sync_copy(trows, acc.at[tdst], add=True)

    wait_scatter(5)
    wait_scatter(6)
    wait_scatter(7)
    plsc.subcore_barrier()

    pltpu.sync_copy(acc.at[pl.ds(r0, ROWS_PER_TILE), :],
                    out_hbm.at[c, pl.ds(r0, ROWS_PER_TILE), :])

  return agg_kernel(x, src, dst)


BLK = 2000


def _mlp_block(x_ref, p0_ref, p1_ref, w1_ref, b1_ref, w2_ref, b2_ref,
               out_ref):
  h = x_ref[...] + p0_ref[0] + p1_ref[0]
  h = jnp.dot(h, w1_ref[...], preferred_element_type=jnp.float32)
  h = jnp.maximum(h + b1_ref[...], 0.0)
  out_ref[...] = (
      jnp.dot(h, w2_ref[...], preferred_element_type=jnp.float32)
      + b2_ref[...])


def _mlp(x, partials, W1, b1, W2, b2):
  grid = (N_NODES // BLK,)
  row_spec = pl.BlockSpec((BLK, D), lambda i: (i, 0))
  p0_spec = pl.BlockSpec((1, BLK, D), lambda i: (0, i, 0))
  p1_spec = pl.BlockSpec((1, BLK, D), lambda i: (1, i, 0))
  full = pl.BlockSpec((D, D), lambda i: (0, 0))
  vec = pl.BlockSpec((1, D), lambda i: (0, 0))
  return pl.pallas_call(
      _mlp_block,
      grid=grid,
      in_specs=[row_spec, p0_spec, p1_spec, full, vec, full, vec],
      out_specs=row_spec,
      out_shape=jax.ShapeDtypeStruct((N_NODES, D), jnp.float32),
  )(x, partials, partials, W1, b1.reshape(1, D), W2, b2.reshape(1, D))


@jax.jit
def kernel(x, edge_index, W1, b1, W2, b2):
  src = edge_index[0].astype(jnp.int32)
  dst = edge_index[1].astype(jnp.int32)
  partials = _sc_aggregate(x, src, dst)
  return _mlp(x, partials, W1, b1, W2, b2)


# speedup vs baseline: 15.2846x; 1.0653x over previous
import functools

import jax
import jax.numpy as jnp
from jax import lax
from jax.experimental import pallas as pl
from jax.experimental.pallas import tpu as pltpu
from jax.experimental.pallas import tpu_sc as plsc

N_NODES = 10000
D = 128
N_EDGES = 320000

NC = 2
NS = 16
NW = NC * NS

E_PER_TILE = N_EDGES // NW
CHUNK = 32
N_CHUNKS = 312
TAIL = E_PER_TILE - N_CHUNKS * CHUNK

N_PAD = 10240
ROWS_PER_TILE = N_PAD // NS

NBUF = 8


def _sc_aggregate(x, src, dst):
  mesh = plsc.VectorSubcoreMesh(
      core_axis_name="c", subcore_axis_name="s", num_cores=NC,
      num_subcores=NS)

  @functools.partial(
      pl.kernel,
      out_type=jax.ShapeDtypeStruct((NC, N_PAD, D), jnp.float32),
      mesh=mesh,
      scratch_types=[
          pltpu.VMEM((E_PER_TILE,), jnp.int32),
          [pltpu.VMEM((CHUNK,), jnp.int32) for _ in range(NBUF)],
          [pltpu.VMEM((CHUNK, D), jnp.float32) for _ in range(NBUF)],
          pltpu.VMEM((TAIL,), jnp.int32),
          pltpu.VMEM((TAIL,), jnp.int32),
          pltpu.VMEM((TAIL, D), jnp.float32),
          pltpu.VMEM_SHARED((N_PAD, D), jnp.float32),
          pltpu.SemaphoreType.DMA,
          pltpu.SemaphoreType.DMA,
          [pltpu.SemaphoreType.DMA for _ in range(NBUF)],
          [pltpu.SemaphoreType.DMA for _ in range(NBUF)],
          [pltpu.SemaphoreType.DMA for _ in range(NBUF)],
      ],
  )
  def agg_kernel(x_hbm, src_hbm, dst_hbm, out_hbm,
                 src_all, dst_slot, rows, tsrc, tdst, trows, acc,
                 sem_i, sem_z, sem_d, sem_g, sem_s):
    c = lax.axis_index("c")
    s = lax.axis_index("s")
    wid = s * NC + c
    base = wid * E_PER_TILE
    tail0 = base + N_CHUNKS * CHUNK

    pltpu.async_copy(src_hbm.at[pl.ds(base, E_PER_TILE)], src_all, sem_i)
    pltpu.async_copy(src_hbm.at[pl.ds(tail0, TAIL)], tsrc, sem_i)
    pltpu.async_copy(dst_hbm.at[pl.ds(tail0, TAIL)], tdst, sem_i)

    zeros16 = jnp.zeros((16,), jnp.float32)

    def zrow(i, _):
      for j in range(D // 16):
        rows[7][i, pl.ds(j * 16, 16)] = zeros16
      return 0

    lax.fori_loop(0, CHUNK, zrow, 0)
    r0 = s * ROWS_PER_TILE
    for k in range(ROWS_PER_TILE // CHUNK):
      pltpu.async_copy(rows[7], acc.at[pl.ds(r0 + k * CHUNK, CHUNK), :],
                       sem_z)

    def load_dst(i, sl):
      pltpu.async_copy(dst_hbm.at[pl.ds(base + i * CHUNK, CHUNK)],
                       dst_slot[sl], sem_d[sl])

    def wait_dst(sl):
      pltpu.make_async_copy(dst_hbm.at[pl.ds(base, CHUNK)],
                            dst_slot[sl], sem_d[sl]).wait()

    def start_gather(i, rb):
      pltpu.async_copy(
          x_hbm.at[src_all.at[pl.ds(i * CHUNK, CHUNK)]], rows[rb],
          sem_g[rb])

    def wait_gather(rb):
      pltpu.make_async_copy(
          x_hbm.at[src_all.at[pl.ds(0, CHUNK)]], rows[rb],
          sem_g[rb]).wait()

    def start_scatter(rb, sl):
      pltpu.async_copy(rows[rb], acc.at[dst_slot[sl]], sem_s[rb],
                       add=True)

    def wait_scatter(rb):
      pltpu.make_async_copy(rows[rb], acc.at[dst_slot[0]],
                            sem_s[rb]).wait()

    pltpu.make_async_copy(src_hbm.at[pl.ds(base, E_PER_TILE)], src_all,
                          sem_i).wait()
    for sl in range(4):
      load_dst(sl, sl)
    for bb in range(6):
      start_gather(bb, bb)

    for k in range(ROWS_PER_TILE // CHUNK):
      pltpu.make_async_copy(rows[7],
                            acc.at[pl.ds(r0, CHUNK), :], sem_z).wait()
    plsc.subcore_barrier()

    NG = N_CHUNKS // NBUF
    LAST = NG - 1

    def oct_body(g, _):
      for b in range(NBUF):
        i = g * NBUF + b
        bn = (b + 6) % NBUF
        wait_gather(b)
        wait_dst(b)
        start_scatter(b, b)
        if b >= 2:
          wait_scatter(bn)
        else:
          @pl.when(g > 0)
          def _():
            wait_scatter(bn)
        if b <= 1:
          start_gather(i + 6, bn)
        else:
          @pl.when(g < LAST)
          def _():
            start_gather(i + 6, bn)
        if b <= 3:
          load_dst(i + 4, (b + 4) % NBUF)
        else:
          @pl.when(g < LAST)
          def _():
            load_dst(i + 4, (b + 4) % NBUF)
      return 0

    lax.fori_loop(0, NG, oct_body, 0)

    pltpu.make_async_copy(src_hbm.at[pl.ds(tail0, TAIL)], tsrc,
                          sem_i).wait()
    pltpu.make_async_copy(dst_hbm.at[pl.ds(tail0, TAIL)], tdst,
                          sem_i).wait()
    pltpu.sync_copy(x_hbm.at[tsrc], trows)
    pltpu.sync_copy(trows, acc.at[tdst], add=True)

    wait_scatter(6)
    wait_scatter(7)
    plsc.subcore_barrier()

    pltpu.sync_copy(acc.at[pl.ds(r0, ROWS_PER_TILE), :],
                    out_hbm.at[c, pl.ds(r0, ROWS_PER_TILE), :])

  return agg_kernel(x, src, dst)


BLK = 2000


def _mlp_block(x_ref, p0_ref, p1_ref, w1_ref, b1_ref, w2_ref, b2_ref,
               out_ref):
  h = x_ref[...] + p0_ref[0] + p1_ref[0]
  h = jnp.dot(h, w1_ref[...], preferred_element_type=jnp.float32)
  h = jnp.maximum(h + b1_ref[...], 0.0)
  out_ref[...] = (
      jnp.dot(h, w2_ref[...], preferred_element_type=jnp.float32)
      + b2_ref[...])


def _mlp(x, partials, W1, b1, W2, b2):
  grid = (N_NODES // BLK,)
  row_spec = pl.BlockSpec((BLK, D), lambda i: (i, 0))
  p0_spec = pl.BlockSpec((1, BLK, D), lambda i: (0, i, 0))
  p1_spec = pl.BlockSpec((1, BLK, D), lambda i: (1, i, 0))
  full = pl.BlockSpec((D, D), lambda i: (0, 0))
  vec = pl.BlockSpec((1, D), lambda i: (0, 0))
  return pl.pallas_call(
      _mlp_block,
      grid=grid,
      in_specs=[row_spec, p0_spec, p1_spec, full, vec, full, vec],
      out_specs=row_spec,
      out_shape=jax.ShapeDtypeStruct((N_NODES, D), jnp.float32),
  )(x, partials, partials, W1, b1.reshape(1, D), W2, b2.reshape(1, D))


@jax.jit
def kernel(x, edge_index, W1, b1, W2, b2):
  src = edge_index[0].astype(jnp.int32)
  dst = edge_index[1].astype(jnp.int32)
  partials = _sc_aggregate(x, src, dst)
  return _mlp(x, partials, W1, b1, W2, b2)
